# baseline (speedup 1.0000x reference)
import jax
import jax.numpy as jnp
from jax import lax
from jax.experimental import pallas as pl

E = 8
_GELU_C = 0.7978845608028654


def _gelu_tanh(x):
    return 0.5 * x * (1.0 + jnp.tanh(_GELU_C * (x + 0.044715 * x * x * x)))


def _router_kernel(flat_ref, rw_ref, comb_ref, xb_ref):
    logits = lax.dot_general(
        rw_ref[...], flat_ref[...], (((1,), (1,)), ((), ())),
        preferred_element_type=jnp.float32)
    e, t = logits.shape
    ids = lax.broadcasted_iota(jnp.int32, (e, t), 0)
    m1 = jnp.max(logits, axis=0, keepdims=True)
    a1 = jnp.min(jnp.where(logits == m1, ids, e), axis=0, keepdims=True)
    l2 = jnp.where(ids == a1, -jnp.inf, logits)
    m2 = jnp.max(l2, axis=0, keepdims=True)
    a2 = jnp.min(jnp.where(l2 == m2, ids, e), axis=0, keepdims=True)
    e2 = jnp.exp(m2 - m1)
    p1 = 1.0 / (1.0 + e2)
    p2 = e2 * p1
    comb_ref[...] = jnp.where(ids == a1, p1, 0.0) + jnp.where(ids == a2, p2, 0.0)
    xb_ref[...] = flat_ref[...].astype(jnp.bfloat16)


def _moe_dense_kernel(comb_ref, xb_ref, w1_ref, w2_ref, out_ref):
    e = pl.program_id(0)
    x = xb_ref[...]
    c = comb_ref[0]
    dff = w1_ref.shape[2]
    hf = dff // 2
    hout = w2_ref.shape[2]
    hh = hout // 2

    hs = []
    for lo in (0, hf):
        w1h = w1_ref[0, :, pl.ds(lo, hf)].astype(jnp.bfloat16)
        g = jnp.dot(x, w1h, preferred_element_type=jnp.float32)
        hs.append((_gelu_tanh(g) * c).astype(jnp.bfloat16))

    for ho in (0, hh):
        w2a = w2_ref[0, pl.ds(0, hf), pl.ds(ho, hh)].astype(jnp.bfloat16)
        w2b = w2_ref[0, pl.ds(hf, hf), pl.ds(ho, hh)].astype(jnp.bfloat16)
        y = (jnp.dot(hs[0], w2a, preferred_element_type=jnp.float32)
             + jnp.dot(hs[1], w2b, preferred_element_type=jnp.float32))

        @pl.when(e == 0)
        def _(y=y, ho=ho):
            out_ref[:, pl.ds(ho, hh)] = y

        @pl.when(e != 0)
        def _(y=y, ho=ho):
            out_ref[:, pl.ds(ho, hh)] += y


def kernel(hidden_states, router_weight, w1, w2):
    b, s, h = hidden_states.shape
    t = b * s
    dff = w1.shape[2]
    flat = hidden_states.reshape(t, h)

    comb, xb = pl.pallas_call(
        _router_kernel,
        out_shape=(jax.ShapeDtypeStruct((E, t), jnp.float32),
                   jax.ShapeDtypeStruct((t, h), jnp.bfloat16)),
    )(flat, router_weight)
    comb = comb.reshape(E, t, 1)

    out = pl.pallas_call(
        _moe_dense_kernel,
        grid=(E,),
        in_specs=[
            pl.BlockSpec((1, t, 1), lambda e: (e, 0, 0)),
            pl.BlockSpec((t, h), lambda e: (0, 0)),
            pl.BlockSpec((1, h, dff), lambda e: (e, 0, 0)),
            pl.BlockSpec((1, dff, h), lambda e: (e, 0, 0)),
        ],
        out_specs=pl.BlockSpec((t, h), lambda e: (0, 0)),
        out_shape=jax.ShapeDtypeStruct((t, h), jnp.float32),
    )(comb, xb, w1, w2)
    return out.reshape(b, s, h)


# speedup vs baseline: 1.0151x; 1.0151x over previous
import jax
import jax.numpy as jnp
from jax import lax
from jax.experimental import pallas as pl

E = 8
_GELU_C = 0.7978845608028654


def _gelu_tanh(x):
    return 0.5 * x * (1.0 + jnp.tanh(_GELU_C * (x + 0.044715 * x * x * x)))


def _router_kernel(flat_ref, rw_ref, comb_ref, xb_ref):
    logits = lax.dot_general(
        rw_ref[...], flat_ref[...], (((1,), (1,)), ((), ())),
        preferred_element_type=jnp.float32)
    e, t = logits.shape
    ids = lax.broadcasted_iota(jnp.int32, (e, t), 0)
    m1 = jnp.max(logits, axis=0, keepdims=True)
    a1 = jnp.min(jnp.where(logits == m1, ids, e), axis=0, keepdims=True)
    l2 = jnp.where(ids == a1, -jnp.inf, logits)
    m2 = jnp.max(l2, axis=0, keepdims=True)
    a2 = jnp.min(jnp.where(l2 == m2, ids, e), axis=0, keepdims=True)
    e2 = jnp.exp(m2 - m1)
    p1 = 1.0 / (1.0 + e2)
    p2 = e2 * p1
    comb_ref[...] = jnp.where(ids == a1, p1, 0.0) + jnp.where(ids == a2, p2, 0.0)
    xb_ref[...] = flat_ref[...].astype(jnp.bfloat16)


def _moe_dense_kernel(comb_ref, xb_ref, w1_ref, w2_ref, out_ref):
    e = pl.program_id(0)
    x = xb_ref[...]
    dff = w1_ref.shape[2]
    hf = dff // 2

    def half(lo):
        w1 = w1_ref[0, :, pl.ds(lo, hf)].astype(jnp.bfloat16)
        h = jnp.dot(x, w1, preferred_element_type=jnp.float32)
        h = _gelu_tanh(h).astype(jnp.bfloat16)
        w2 = w2_ref[0, pl.ds(lo, hf), :].astype(jnp.bfloat16)
        return jnp.dot(h, w2, preferred_element_type=jnp.float32)

    y = half(0) + half(hf)
    contrib = y * comb_ref[0]

    @pl.when(e == 0)
    def _():
        out_ref[...] = contrib

    @pl.when(e != 0)
    def _():
        out_ref[...] += contrib


def kernel(hidden_states, router_weight, w1, w2):
    b, s, h = hidden_states.shape
    t = b * s
    dff = w1.shape[2]
    flat = hidden_states.reshape(t, h)

    comb, xb = pl.pallas_call(
        _router_kernel,
        out_shape=(jax.ShapeDtypeStruct((E, t), jnp.float32),
                   jax.ShapeDtypeStruct((t, h), jnp.bfloat16)),
    )(flat, router_weight)
    comb = comb.reshape(E, t, 1)

    out = pl.pallas_call(
        _moe_dense_kernel,
        grid=(E,),
        in_specs=[
            pl.BlockSpec((1, t, 1), lambda e: (e, 0, 0)),
            pl.BlockSpec((t, h), lambda e: (0, 0)),
            pl.BlockSpec((1, h, dff), lambda e: (e, 0, 0)),
            pl.BlockSpec((1, dff, h), lambda e: (e, 0, 0)),
        ],
        out_specs=pl.BlockSpec((t, h), lambda e: (0, 0)),
        out_shape=jax.ShapeDtypeStruct((t, h), jnp.float32),
    )(comb, xb, w1, w2)
    return out.reshape(b, s, h)
